# baseline (speedup 1.0000x reference)
import jax
import jax.numpy as jnp
from jax import lax
from jax.experimental import pallas as pl
from jax.experimental.pallas import tpu as pltpu
from jax.experimental.pallas import tpu_sc as plsc

BATCH = 16384
D = 64
V = 1000000
NW = 32
SLICE = 31232
VC = 512
NWIN = SLICE // VC
ROWS_PAD = BATCH + 16
OCAP = 1024
IDXC = 2048
NIDXC = BATCH // IDXC
MCAP = 64
BPW = BATCH // NW
PITCH = 129
RCH = 128


def _iota16():
    return lax.iota(jnp.int32, 16)


def _extract_body(user_h, pos_h, neg_h, utt_h, itt_h,
                  urows_h, prows_h, nrows_h,
                  idxbuf0, idxbuf1, oeid_u, ovl_u, oeid_p, ovl_p,
                  oeid_n, ovl_n, wbuf0, wbuf1, tbuf, meid, moff,
                  srows_u, srows_p, srows_n,
                  sem, semw):
    wid = lax.axis_index("s") * 2 + lax.axis_index("c")
    w0 = wid * SLICE
    is_last = wid == NW - 1
    w_end = jnp.where(is_last, V, w0 + SLICE)
    kc = jnp.where(is_last, NWIN + 1, NWIN)
    zero16 = jnp.zeros((16,), jnp.int32)

    sk0 = pl.multiple_of(w0, 128)
    pltpu.async_copy(utt_h.at[:, pl.ds(sk0, VC)], wbuf0, semw)

    def scan_stream(src_h, oeid, ovl):
        pltpu.async_copy(src_h.at[pl.ds(0, IDXC)], idxbuf0, sem)

        def chunk2(ci, cnt):
            for par in range(2):
                c = 2 * ci + par
                ib = idxbuf0 if par == 0 else idxbuf1
                nbuf = idxbuf1 if par == 0 else idxbuf0
                pltpu.make_async_copy(
                    src_h.at[pl.ds(0, IDXC)], ib, sem).wait()

                @pl.when(c + 1 < NIDXC)
                def _(c=c, nbuf=nbuf):
                    pltpu.async_copy(
                        src_h.at[pl.ds((c + 1) * IDXC, IDXC)], nbuf, sem)

                def blk(b, cnt, ib=ib, c=c):
                    off = pl.multiple_of(b * 16, 16)
                    v = ib[pl.ds(off, 16)]
                    m = (v >= w0) & (v < w_end)
                    mi = m.astype(jnp.int32)
                    pos = cnt + plsc.cumsum(mi) - mi
                    m = m & (pos < OCAP)
                    eid = c * IDXC + b * 16 + _iota16()
                    plsc.store_scatter(oeid, [pos], eid, mask=m)
                    plsc.store_scatter(ovl, [pos], v - w0, mask=m)
                    return cnt + plsc.all_reduce_population_count(m)

                cnt = lax.fori_loop(0, IDXC // 16, blk, cnt)
            return cnt

        cnt = lax.fori_loop(0, NIDXC // 2, chunk2, zero16)
        return jnp.max(cnt)

    no_u = scan_stream(user_h, oeid_u, ovl_u)
    no_p = scan_stream(pos_h, oeid_p, ovl_p)
    no_n = scan_stream(neg_h, oeid_n, ovl_n)

    def extract(buf, k, oeid, ovl, no, dst_h, srows):
        for q in range(4):
            meid[pl.ds(q * 16, 16)] = jnp.full((16,), ROWS_PAD - 16,
                                               jnp.int32)
            moff[pl.ds(q * 16, 16)] = jnp.zeros((16,), jnp.int32)

        def blk(b, mc):
            off = pl.multiple_of(b * 16, 16)
            ov = ovl[pl.ds(off, 16)]
            oe = oeid[pl.ds(off, 16)]
            valid = (b * 16 + _iota16()) < no
            m = valid & ((ov >> 9) == k)
            mi = m.astype(jnp.int32)
            pos = mc + plsc.cumsum(mi) - mi
            m = m & (pos < MCAP)
            plsc.store_scatter(meid, [pos], oe, mask=m)
            plsc.store_scatter(moff, [pos], ov & 511, mask=m)
            return mc + plsc.all_reduce_population_count(m)

        nblk = (no + 15) >> 4
        mc = lax.fori_loop(0, nblk, blk, zero16)
        nm = jnp.max(mc)

        for b in range(4):
            @pl.when(b * 16 < nm)
            def _(b=b):
                offs = moff[pl.ds(b * 16, 16)]
                jr = b * 16 + _iota16()

                def dcol(d, carry):
                    ds = jnp.full((16,), d, jnp.int32)
                    vals = plsc.load_gather(buf, [ds, offs])
                    plsc.store_scatter(srows, [jr, ds], vals)
                    return carry

                lax.fori_loop(0, D, dcol, 0)
                ids = plsc.load_gather(meid, [jr])
                pltpu.async_copy(
                    srows.at[pl.ds(b * 16, 16), :], dst_h.at[ids], sem
                )
        return nm

    def sweep(tbl_h, streams, primed):
        def fire(k, wb):
            sk = pl.multiple_of(w0 + k * VC, 128)
            pltpu.async_copy(tbl_h.at[:, pl.ds(sk, VC)], wb, semw)

        def wait_w(wb):
            pltpu.make_async_copy(
                tbl_h.at[:, pl.ds(0, VC)], wb, semw).wait()

        if not primed:
            fire(0, wbuf0)

        def body(i, carry):
            for par in range(2):
                k = 2 * i + par
                wb = wbuf0 if par == 0 else wbuf1
                nb = wbuf1 if par == 0 else wbuf0

                @pl.when(k < kc)
                def _(k=k, wb=wb, nb=nb):
                    wait_w(wb)

                    @pl.when(k + 1 < kc)
                    def _():
                        fire(k + 1, nb)

                    for (oeid, ovl, no, dst_h, srows) in streams:
                        nm = extract(wb, k, oeid, ovl, no, dst_h, srows)
                        for b in range(4):
                            @pl.when(b * 16 < nm)
                            def _(b=b, srows=srows):
                                pltpu.make_async_copy(
                                    urows_h.at[pl.ds(0, 16), :],
                                    srows.at[pl.ds(b * 16, 16), :], sem
                                ).wait()
            return carry

        lax.fori_loop(0, (NWIN + 2) // 2, body, 0)

    sweep(utt_h, ((oeid_u, ovl_u, no_u, urows_h, srows_u),), True)
    sweep(itt_h, ((oeid_p, ovl_p, no_p, prows_h, srows_p),
                  (oeid_n, ovl_n, no_n, nrows_h, srows_n)), False)

    @pl.when(is_last)
    def _():
        for (oeid, ovl, no, dst_h, tbl_h) in (
            (oeid_u, ovl_u, no_u, urows_h, utt_h),
            (oeid_p, ovl_p, no_p, prows_h, itt_h),
            (oeid_n, ovl_n, no_n, nrows_h, itt_h),
        ):
            pltpu.sync_copy(tbl_h.at[:, pl.ds(V - 64, 64)], tbuf)
            def blk(b, mc, oeid=oeid, ovl=ovl, no=no):
                off = pl.multiple_of(b * 16, 16)
                ov = ovl[pl.ds(off, 16)]
                oe = oeid[pl.ds(off, 16)]
                valid = (b * 16 + _iota16()) < no
                m = valid & ((ov >> 9) == NWIN + 1)
                mi = m.astype(jnp.int32)
                pos = mc + plsc.cumsum(mi) - mi
                m = m & (pos < MCAP)
                plsc.store_scatter(meid, [pos], oe, mask=m)
                plsc.store_scatter(moff, [pos], ov - (NWIN + 1) * VC,
                                   mask=m)
                return mc + plsc.all_reduce_population_count(m)

            for q in range(4):
                meid[pl.ds(q * 16, 16)] = jnp.full((16,), ROWS_PAD - 16,
                                                   jnp.int32)
                moff[pl.ds(q * 16, 16)] = jnp.zeros((16,), jnp.int32)
            mc = lax.fori_loop(0, (no + 15) >> 4, blk, zero16)
            nm = jnp.max(mc)

            for b in range(4):
                @pl.when(b * 16 < nm)
                def _(b=b, dst_h=dst_h):
                    offs = moff[pl.ds(b * 16, 16)]
                    jr = b * 16 + _iota16()

                    def dcol(d, carry):
                        ds = jnp.full((16,), d, jnp.int32)
                        vals = plsc.load_gather(tbuf, [ds, offs])
                        plsc.store_scatter(srows_u, [jr, ds], vals)
                        return carry

                    lax.fori_loop(0, D, dcol, 0)
                    ids = plsc.load_gather(meid, [jr])
                    pltpu.async_copy(
                        srows_u.at[pl.ds(b * 16, 16), :], dst_h.at[ids],
                        sem
                    ).wait()


def _dot_body(urows_h, prows_h, nrows_h, outp_h, outn_h,
              ub, pb, nb, outp_v, outn_v, sem):
    wid = lax.axis_index("s") * 2 + lax.axis_index("c")
    base = wid * BPW

    def chunk(c, carry):
        r0 = pl.multiple_of(base + c * RCH, 8)
        copies = []
        for src_h, buf in ((urows_h, ub), (prows_h, pb), (nrows_h, nb)):
            copies.append(pltpu.async_copy(
                src_h.at[pl.ds(r0, RCH), :], buf.at[:, pl.ds(0, 128)],
                sem))
        for cp in copies:
            cp.wait()

        def group(g, carry):
            rows = g * 16 + _iota16()

            def col(d, accs):
                ap, an = accs
                cols = jnp.full((16,), d, jnp.int32)
                u = plsc.load_gather(ub, [rows, cols])
                p = plsc.load_gather(pb, [rows, cols])
                n = plsc.load_gather(nb, [rows, cols])
                return ap + u * p, an + u * n

            zero = jnp.zeros((16,), jnp.float32)
            ap, an = lax.fori_loop(0, D, col, (zero, zero))
            o = c * RCH + g * 16
            outp_v[pl.ds(o, 16)] = ap
            outn_v[pl.ds(o, 16)] = an
            return carry

        lax.fori_loop(0, RCH // 16, group, 0)
        return carry

    lax.fori_loop(0, BPW // RCH, chunk, 0)
    pltpu.sync_copy(outp_v, outp_h.at[pl.ds(base, BPW)])
    pltpu.sync_copy(outn_v, outn_h.at[pl.ds(base, BPW)])


@jax.jit
def kernel(user, pos_item, neg_item, user_table, item_table):
    mesh = plsc.VectorSubcoreMesh(core_axis_name="c", subcore_axis_name="s")
    params = pltpu.CompilerParams(needs_layout_passes=False)
    extract = pl.kernel(
        _extract_body,
        mesh=mesh,
        compiler_params=params,
        out_type=(
            jax.ShapeDtypeStruct((ROWS_PAD, 128), jnp.float32),
            jax.ShapeDtypeStruct((ROWS_PAD, 128), jnp.float32),
            jax.ShapeDtypeStruct((ROWS_PAD, 128), jnp.float32),
        ),
        scratch_types=[
            pltpu.VMEM((IDXC,), jnp.int32),
            pltpu.VMEM((IDXC,), jnp.int32),
            pltpu.VMEM((OCAP,), jnp.int32),
            pltpu.VMEM((OCAP,), jnp.int32),
            pltpu.VMEM((OCAP,), jnp.int32),
            pltpu.VMEM((OCAP,), jnp.int32),
            pltpu.VMEM((OCAP,), jnp.int32),
            pltpu.VMEM((OCAP,), jnp.int32),
            pltpu.VMEM((D, VC), jnp.float32),
            pltpu.VMEM((D, VC), jnp.float32),
            pltpu.VMEM((D, 64), jnp.float32),
            pltpu.VMEM((MCAP,), jnp.int32),
            pltpu.VMEM((MCAP,), jnp.int32),
            pltpu.VMEM((MCAP, 128), jnp.float32),
            pltpu.VMEM((MCAP, 128), jnp.float32),
            pltpu.VMEM((MCAP, 128), jnp.float32),
            pltpu.SemaphoreType.DMA,
            pltpu.SemaphoreType.DMA,
        ],
    )
    dot = pl.kernel(
        _dot_body,
        mesh=mesh,
        compiler_params=params,
        out_type=(
            jax.ShapeDtypeStruct((BATCH,), jnp.float32),
            jax.ShapeDtypeStruct((BATCH,), jnp.float32),
        ),
        scratch_types=[
            pltpu.VMEM((RCH, PITCH), jnp.float32),
            pltpu.VMEM((RCH, PITCH), jnp.float32),
            pltpu.VMEM((RCH, PITCH), jnp.float32),
            pltpu.VMEM((BPW,), jnp.float32),
            pltpu.VMEM((BPW,), jnp.float32),
            pltpu.SemaphoreType.DMA,
        ],
    )
    urows, prows, nrows = extract(
        user, pos_item, neg_item, user_table.T, item_table.T
    )
    return dot(urows, prows, nrows)


# speedup vs baseline: 1.6326x; 1.6326x over previous
import functools

import jax
import jax.numpy as jnp
from jax import lax
from jax.experimental import pallas as pl
from jax.experimental.pallas import tpu as pltpu
from jax.experimental.pallas import tpu_sc as plsc

BATCH = 16384
D = 64
NC = 2
NS = 16
NW = NC * NS
BPW = BATCH // NW
CHUNK = 128
NCHUNK = BPW // CHUNK


def _bpr_body(user_h, pos_h, neg_h, ut_h, it_h, outp_h, outn_h,
              uidx, pidx, nidx, urows, prows, nrows, outp_v, outn_v, sem):
    wid = lax.axis_index("s") * NC + lax.axis_index("c")
    base = wid * BPW

    pltpu.sync_copy(user_h.at[pl.ds(base, BPW)], uidx)
    pltpu.sync_copy(pos_h.at[pl.ds(base, BPW)], pidx)
    pltpu.sync_copy(neg_h.at[pl.ds(base, BPW)], nidx)

    copies = []
    for j in range(NCHUNK):
        sl = pl.ds(j * CHUNK, CHUNK)
        copies.append(pltpu.async_copy(ut_h.at[uidx.at[sl]], urows.at[sl], sem))
        copies.append(pltpu.async_copy(it_h.at[pidx.at[sl]], prows.at[sl], sem))
        copies.append(pltpu.async_copy(it_h.at[nidx.at[sl]], nrows.at[sl], sem))
    for c in copies:
        c.wait()

    def group(g, carry):
        rows = g * 16 + lax.iota(jnp.int32, 16)

        def col(c, accs):
            ap, an = accs
            cols = jnp.full((16,), c, jnp.int32)
            u = plsc.load_gather(urows, [rows, cols])
            p = plsc.load_gather(prows, [rows, cols])
            n = plsc.load_gather(nrows, [rows, cols])
            return ap + u * p, an + u * n

        zero = jnp.zeros((16,), jnp.float32)
        ap, an = lax.fori_loop(0, D, col, (zero, zero))
        outp_v[pl.ds(g * 16, 16)] = ap
        outn_v[pl.ds(g * 16, 16)] = an
        return carry

    lax.fori_loop(0, BPW // 16, group, 0)

    pltpu.sync_copy(outp_v, outp_h.at[pl.ds(base, BPW)])
    pltpu.sync_copy(outn_v, outn_h.at[pl.ds(base, BPW)])


@jax.jit
def kernel(user, pos_item, neg_item, user_table, item_table):
    mesh = plsc.VectorSubcoreMesh(core_axis_name="c", subcore_axis_name="s")
    f = pl.kernel(
        _bpr_body,
        mesh=mesh,
        compiler_params=pltpu.CompilerParams(
            needs_layout_passes=False, use_tc_tiling_on_sc=False),
        out_type=(
            jax.ShapeDtypeStruct((BATCH,), jnp.float32),
            jax.ShapeDtypeStruct((BATCH,), jnp.float32),
        ),
        scratch_types=[
            pltpu.VMEM((BPW,), jnp.int32),
            pltpu.VMEM((BPW,), jnp.int32),
            pltpu.VMEM((BPW,), jnp.int32),
            pltpu.VMEM((BPW, D), jnp.float32),
            pltpu.VMEM((BPW, D), jnp.float32),
            pltpu.VMEM((BPW, D), jnp.float32),
            pltpu.VMEM((BPW,), jnp.float32),
            pltpu.VMEM((BPW,), jnp.float32),
            pltpu.SemaphoreType.DMA,
        ],
    )
    return f(user, pos_item, neg_item, user_table, item_table)
